# baseline (speedup 1.0000x reference)
import functools

import jax
import jax.numpy as jnp
from jax import lax
from jax.experimental import pallas as pl
from jax.experimental.pallas import tpu as pltpu
from jax.experimental.pallas import tpu_sc as plsc

N = 50000
E = 800000
B = 512
EMB = 512
HID = 64

NC = 2
NS = 16
NW = NC * NS
CHUNK = 128
PER_W = 25088
E_PAD = PER_W * NW
NCHUNK = PER_W // CHUNK
NPAD = 50432
RPT = NPAD // NS
ZROWS = 394
YROWS = 50016
BLK = 2000
GRID = N // BLK

_f32 = jnp.float32
_i32 = jnp.int32


KB = 2
G = NCHUNK // KB
TOT_CHUNKS = E_PAD // CHUNK


def _deg_body(idx2_hbm, deg0_hbm, deg1_hbm, ones_v, idx_v, zbuf_v, acc_sh, sem_i):
    c = lax.axis_index("c")
    s = lax.axis_index("s")

    def _init(i, carry):
        ones_v[pl.ds(i * 16, 16)] = jnp.ones((16,), _f32)
        return carry
    lax.fori_loop(0, CHUNK // 16, _init, 0)

    def _zb(i, carry):
        zbuf_v[pl.ds(i * 16, 16)] = jnp.zeros((16,), _f32)
        return carry
    lax.fori_loop(0, RPT // 16, _zb, 0)
    pltpu.sync_copy(zbuf_v, acc_sh.at[pl.ds(s * RPT, RPT)])
    plsc.subcore_barrier()

    cbase = (s * NC + c) * NCHUNK

    def _start_idx(g, slot):
        pltpu.async_copy(idx2_hbm.at[pl.ds(cbase + g * KB, KB)],
                         idx_v.at[slot], sem_i.at[slot])

    def _wait_idx(slot):
        pltpu.make_async_copy(idx2_hbm.at[pl.ds(0, KB)],
                              idx_v.at[slot], sem_i.at[slot]).wait()

    _start_idx(0, 0)
    _start_idx(1, 1)
    _wait_idx(0)

    def _group(g, carry):
        slot = lax.rem(g, 2)
        nslot = lax.rem(g + 1, 2)
        for b in range(KB):
            pltpu.sync_copy(ones_v, acc_sh.at[idx_v.at[slot, b, 1]], add=True)

        @pl.when(g + 2 < G)
        def _pref():
            _start_idx(g + 2, slot)

        @pl.when(g + 1 < G)
        def _nw():
            _wait_idx(nslot)
        return carry
    lax.fori_loop(0, G, _group, 0)
    plsc.subcore_barrier()

    pltpu.sync_copy(acc_sh.at[pl.ds(s * RPT, RPT)], zbuf_v)

    @pl.when(c == 0)
    def _w0():
        pltpu.sync_copy(zbuf_v, deg0_hbm.at[pl.ds(s * RPT, RPT)])

    @pl.when(c == 1)
    def _w1():
        pltpu.sync_copy(zbuf_v, deg1_hbm.at[pl.ds(s * RPT, RPT)])


def _deg_pass(idx2):
    mesh = plsc.VectorSubcoreMesh(core_axis_name="c", subcore_axis_name="s")
    return pl.kernel(
        _deg_body,
        out_type=[jax.ShapeDtypeStruct((NPAD,), _f32),
                  jax.ShapeDtypeStruct((NPAD,), _f32)],
        mesh=mesh,
        scratch_types=[
            pltpu.VMEM((CHUNK,), _f32),
            pltpu.VMEM((2, KB, 2, CHUNK), _i32),
            pltpu.VMEM((RPT,), _f32),
            pltpu.VMEM_SHARED((NPAD,), _f32),
            pltpu.SemaphoreType.DMA((2,)),
        ],
        compiler_params=pltpu.CompilerParams(use_tc_tiling_on_sc=False),
    )(idx2)


def _conv_body(ylo_hbm, yhi_hbm, idx2_hbm, outlo_hbm, outhi_hbm,
               idx_v, rows_v, zbuf_v, acc_sh, sem_i, sem_g):
    c = lax.axis_index("c")
    s = lax.axis_index("s")

    def _zb(i, carry):
        r = i // 2
        zbuf_v[r, pl.ds((i % 2) * 16, 16)] = jnp.zeros((16,), _f32)
        return carry
    lax.fori_loop(0, ZROWS * 2, _zb, 0)

    def _zacc(i, carry):
        pltpu.sync_copy(zbuf_v, acc_sh.at[pl.ds(s * RPT + i * ZROWS, ZROWS)])
        return carry
    lax.fori_loop(0, RPT // ZROWS, _zacc, 0)
    plsc.subcore_barrier()

    cbase = (s * NC + c) * NCHUNK

    def _start_idx(g, slot):
        pltpu.async_copy(idx2_hbm.at[pl.ds(cbase + g * KB, KB)],
                         idx_v.at[slot], sem_i.at[slot])

    def _wait_idx(slot):
        pltpu.make_async_copy(idx2_hbm.at[pl.ds(0, KB)],
                              idx_v.at[slot], sem_i.at[slot]).wait()

    def _start_gathers(slot):
        for b in range(KB):
            @pl.when(c == 0)
            def _g0():
                pltpu.async_copy(ylo_hbm.at[idx_v.at[slot, b, 0]],
                                 rows_v.at[slot, b], sem_g.at[slot])

            @pl.when(c == 1)
            def _g1():
                pltpu.async_copy(yhi_hbm.at[idx_v.at[slot, b, 0]],
                                 rows_v.at[slot, b], sem_g.at[slot])

    def _wait_gathers(slot):
        for b in range(KB):
            pltpu.make_async_copy(ylo_hbm.at[pl.ds(0, CHUNK)],
                                  rows_v.at[slot, b], sem_g.at[slot]).wait()

    _start_idx(0, 0)
    _start_idx(1, 1)
    _wait_idx(0)
    _start_gathers(0)

    def _group(g, carry):
        slot = lax.rem(g, 2)
        nslot = lax.rem(g + 1, 2)
        _wait_gathers(slot)

        @pl.when(g + 1 < G)
        def _next():
            _wait_idx(nslot)
            _start_gathers(nslot)

        for b in range(KB):
            pltpu.sync_copy(rows_v.at[slot, b],
                            acc_sh.at[idx_v.at[slot, b, 1]], add=True)

        @pl.when(g + 2 < G)
        def _pref():
            _start_idx(g + 2, slot)
        return carry
    lax.fori_loop(0, G, _group, 0)
    plsc.subcore_barrier()

    def _wout(i, carry):
        r = s * RPT + i * ZROWS
        pltpu.sync_copy(acc_sh.at[pl.ds(r, ZROWS)], zbuf_v)

        @pl.when(c == 0)
        def _o0():
            pltpu.sync_copy(zbuf_v, outlo_hbm.at[pl.ds(r, ZROWS)])

        @pl.when(c == 1)
        def _o1():
            pltpu.sync_copy(zbuf_v, outhi_hbm.at[pl.ds(r, ZROWS)])
        return carry
    lax.fori_loop(0, RPT // ZROWS, _wout, 0)


def _conv_pass(ylo, yhi, idx2):
    mesh = plsc.VectorSubcoreMesh(core_axis_name="c", subcore_axis_name="s")
    return pl.kernel(
        _conv_body,
        out_type=[jax.ShapeDtypeStruct((NPAD, 32), _f32),
                  jax.ShapeDtypeStruct((NPAD, 32), _f32)],
        mesh=mesh,
        scratch_types=[
            pltpu.VMEM((2, KB, 2, CHUNK), _i32),
            pltpu.VMEM((2, KB, CHUNK, 32), _f32),
            pltpu.VMEM((ZROWS, 32), _f32),
            pltpu.VMEM_SHARED((NPAD, 32), _f32),
            pltpu.SemaphoreType.DMA((2,)),
            pltpu.SemaphoreType.DMA((2,)),
        ],
        compiler_params=pltpu.CompilerParams(use_tc_tiling_on_sc=False),
    )(ylo, yhi, idx2)


CPB = 128


def _idx_body(ei_ref, o_ref):
    i = pl.program_id(0)
    cols = i * (CPB * CHUNK) + lax.broadcasted_iota(_i32, (2, CPB * CHUNK), 1)
    v = jnp.where(cols < E, ei_ref[...], N)
    o_ref[...] = v.reshape(2, CPB, CHUNK).transpose(1, 0, 2)


def _idx_call(edge_index):
    return pl.pallas_call(
        _idx_body,
        grid=(TOT_CHUNKS // CPB,),
        in_specs=[pl.BlockSpec((2, CPB * CHUNK), lambda i: (0, i))],
        out_specs=pl.BlockSpec((CPB, 2, CHUNK), lambda i: (i, 0, 0)),
        out_shape=jax.ShapeDtypeStruct((TOT_CHUNKS, 2, CHUNK), _i32),
    )(edge_index)


def _emb_body(sm_ref, w1_ref, b1_ref, w2_ref, b2_ref, o_ref):
    h = jnp.maximum(
        jnp.dot(sm_ref[...][:, 0, :], w1_ref[...], preferred_element_type=_f32, precision=lax.Precision.HIGHEST)
        + b1_ref[...], 0.0)
    o_ref[...] = jnp.dot(h, w2_ref[...], preferred_element_type=_f32, precision=lax.Precision.HIGHEST) + b2_ref[...]


def _emb_call(sm, W_e1, b_e1, W_e2, b_e2):
    return pl.pallas_call(
        _emb_body,
        out_shape=jax.ShapeDtypeStruct((B, HID), _f32),
    )(sm, W_e1, b_e1.reshape(1, -1), W_e2, b_e2.reshape(1, -1))


def _rsqrt(v):
    r = lax.rsqrt(v)
    return r * (1.5 - 0.5 * v * r * r)


def _y1_body(d0_ref, d1_ref, x_ref, w_ref, ylo_ref, yhi_ref):
    dis = jnp.transpose(_rsqrt(d0_ref[...][0] + d1_ref[...][0] + 1.0))
    y = dis * jnp.dot(x_ref[...], w_ref[...], preferred_element_type=_f32, precision=lax.Precision.HIGHEST)
    ylo_ref[...] = y[:, :32]
    yhi_ref[...] = y[:, 32:]


def _y1_call(d0, d1, x, W_c1):
    return pl.pallas_call(
        _y1_body,
        grid=(GRID,),
        in_specs=[pl.BlockSpec((1, 1, BLK), lambda i: (i, 0, 0)),
                  pl.BlockSpec((1, 1, BLK), lambda i: (i, 0, 0)),
                  pl.BlockSpec((BLK, 64), lambda i: (i, 0)),
                  pl.BlockSpec((64, 64), lambda i: (0, 0))],
        out_specs=[pl.BlockSpec((BLK, 32), lambda i: (i, 0)),
                   pl.BlockSpec((BLK, 32), lambda i: (i, 0))],
        out_shape=[jax.ShapeDtypeStruct((YROWS, 32), _f32),
                   jax.ShapeDtypeStruct((YROWS, 32), _f32)],
    )(d0, d1, x, W_c1)


def _y2_body(d0_ref, d1_ref, s1lo_ref, s1hi_ref, y1lo_ref, y1hi_ref,
             blo_ref, bhi_ref, y2lo_ref, y2hi_ref):
    dis = jnp.transpose(_rsqrt(d0_ref[...][0] + d1_ref[...][0] + 1.0))
    glo = jnp.maximum(dis * (s1lo_ref[...] + y1lo_ref[...]) + blo_ref[...], 0.0)
    ghi = jnp.maximum(dis * (s1hi_ref[...] + y1hi_ref[...]) + bhi_ref[...], 0.0)
    y2lo_ref[...] = dis * glo
    y2hi_ref[...] = dis * ghi


def _y2_call(d0, d1, s1lo, s1hi, y1lo, y1hi, b_c1):
    row = lambda i: (i, 0)
    return pl.pallas_call(
        _y2_body,
        grid=(GRID,),
        in_specs=[pl.BlockSpec((1, 1, BLK), lambda i: (i, 0, 0)),
                  pl.BlockSpec((1, 1, BLK), lambda i: (i, 0, 0)),
                  pl.BlockSpec((BLK, 32), row),
                  pl.BlockSpec((BLK, 32), row),
                  pl.BlockSpec((BLK, 32), row),
                  pl.BlockSpec((BLK, 32), row),
                  pl.BlockSpec((1, 32), lambda i: (0, 0)),
                  pl.BlockSpec((1, 32), lambda i: (0, 0))],
        out_specs=[pl.BlockSpec((BLK, 32), row),
                   pl.BlockSpec((BLK, 32), row)],
        out_shape=[jax.ShapeDtypeStruct((YROWS, 32), _f32),
                   jax.ShapeDtypeStruct((YROWS, 32), _f32)],
    )(d0, d1, s1lo, s1hi, y1lo, y1hi,
      b_c1[:32].reshape(1, 32), b_c1[32:].reshape(1, 32))


def _head_body(d0_ref, d1_ref, s2lo_ref, s2hi_ref, y2lo_ref, y2hi_ref,
               batch_ref, emb_ref, wc2_ref, bc2_ref, wgf_ref, bgf_ref,
               wa1_ref, ba1_ref, wa2_ref, ba2_ref, wf1_ref, bf1_ref,
               wff_ref, bff_ref, o_ref, acc_ref, cnt_ref):
    i = pl.program_id(0)

    @pl.when(i == 0)
    def _init():
        acc_ref[...] = jnp.zeros_like(acc_ref)
        cnt_ref[...] = jnp.zeros_like(cnt_ref)

    dis_row = _rsqrt(d0_ref[...][0] + d1_ref[...][0] + 1.0)
    t = jnp.concatenate([s2lo_ref[...] + y2lo_ref[...],
                         s2hi_ref[...] + y2hi_ref[...]], axis=1)
    oht = (batch_ref[...][0] ==
           lax.broadcasted_iota(_i32, (B, BLK), 0)).astype(_f32) * dis_row
    acc_ref[...] += lax.dot_general(oht, t, (((1,), (0,)), ((), ())),
                                    preferred_element_type=_f32,
                                    precision=lax.Precision.HIGHEST)
    cnt_ref[...] += jnp.sum((batch_ref[...][0] ==
                             lax.broadcasted_iota(_i32, (B, BLK), 0))
                            .astype(_f32), axis=1, keepdims=True)

    @pl.when(i == GRID - 1)
    def _final():
        cnt = cnt_ref[...]
        pooled64 = acc_ref[...] / jnp.maximum(cnt, 1.0)
        nz = (cnt > 0).astype(_f32)
        pooled = jnp.dot(pooled64, wc2_ref[...],
                         preferred_element_type=_f32, precision=lax.Precision.HIGHEST) + bc2_ref[...] * nz
        gcn = jnp.dot(pooled, wgf_ref[...],
                      preferred_element_type=_f32, precision=lax.Precision.HIGHEST) + bgf_ref[...]
        emb = emb_ref[...]
        comb = jnp.concatenate([emb, gcn], axis=1)
        a = jnp.maximum(jnp.dot(comb, wa1_ref[...],
                                preferred_element_type=_f32, precision=lax.Precision.HIGHEST) + ba1_ref[...], 0.0)
        att = jax.nn.sigmoid(jnp.dot(a, wa2_ref[...],
                                     preferred_element_type=_f32, precision=lax.Precision.HIGHEST) + ba2_ref[...])
        fused = emb * att + gcn * (1.0 - att)
        f1 = jnp.dot(fused, wf1_ref[...],
                     preferred_element_type=_f32, precision=lax.Precision.HIGHEST) + bf1_ref[...]
        o_ref[...] = jnp.dot(f1, wff_ref[...],
                             preferred_element_type=_f32, precision=lax.Precision.HIGHEST) + bff_ref[...]


def _head_call(d0, d1, s2lo, s2hi, y2lo, y2hi, batch2d, emb_out,
               W_c2, b_c2, W_gf, b_gf, W_a1, b_a1, W_a2, b_a2,
               W_f1, b_f1, W_ff, b_ff):
    row = lambda i: (i, 0)
    full = lambda i: (0, 0)
    return pl.pallas_call(
        _head_body,
        grid=(GRID,),
        in_specs=[pl.BlockSpec((1, 1, BLK), lambda i: (i, 0, 0)),
                  pl.BlockSpec((1, 1, BLK), lambda i: (i, 0, 0)),
                  pl.BlockSpec((BLK, 32), row),
                  pl.BlockSpec((BLK, 32), row),
                  pl.BlockSpec((BLK, 32), row),
                  pl.BlockSpec((BLK, 32), row),
                  pl.BlockSpec((1, 1, BLK), lambda i: (i, 0, 0)),
                  pl.BlockSpec((B, HID), full),
                  pl.BlockSpec((HID, 128), full),
                  pl.BlockSpec((1, 128), full),
                  pl.BlockSpec((128, HID), full),
                  pl.BlockSpec((1, HID), full),
                  pl.BlockSpec((2 * HID, HID), full),
                  pl.BlockSpec((1, HID), full),
                  pl.BlockSpec((HID, 1), full),
                  pl.BlockSpec((1, 1), full),
                  pl.BlockSpec((HID, HID), full),
                  pl.BlockSpec((1, HID), full),
                  pl.BlockSpec((HID, 1), full),
                  pl.BlockSpec((1, 1), full)],
        out_specs=pl.BlockSpec((B, 1), full),
        out_shape=jax.ShapeDtypeStruct((B, 1), _f32),
        scratch_shapes=[pltpu.VMEM((B, HID), _f32),
                        pltpu.VMEM((B, 1), _f32)],
    )(d0, d1, s2lo, s2hi, y2lo, y2hi, batch2d, emb_out,
      W_c2, b_c2.reshape(1, -1), W_gf, b_gf.reshape(1, -1),
      W_a1, b_a1.reshape(1, -1), W_a2, b_a2.reshape(1, -1),
      W_f1, b_f1.reshape(1, -1), W_ff, b_ff.reshape(1, -1))


def kernel(smiles_embedding, x, edge_index, batch,
           W_e1, b_e1, W_e2, b_e2, W_c1, b_c1, W_c2, b_c2, W_gf, b_gf,
           W_a1, b_a1, W_a2, b_a2, W_f1, b_f1, W_ff, b_ff):
    idx2 = _idx_call(edge_index.astype(_i32))

    deg0, deg1 = _deg_pass(idx2)
    d0 = deg0[:N].reshape(GRID, 1, BLK)
    d1 = deg1[:N].reshape(GRID, 1, BLK)

    emb_out = _emb_call(smiles_embedding, W_e1, b_e1, W_e2, b_e2)

    y1lo, y1hi = _y1_call(d0, d1, x, W_c1)
    s1lo, s1hi = _conv_pass(y1lo, y1hi, idx2)

    y2lo, y2hi = _y2_call(d0, d1, s1lo, s1hi, y1lo, y1hi, b_c1)
    s2lo, s2hi = _conv_pass(y2lo, y2hi, idx2)

    batch2d = batch.astype(_i32).reshape(GRID, 1, BLK)
    return _head_call(d0, d1, s2lo, s2hi, y2lo, y2hi, batch2d, emb_out,
                      W_c2, b_c2, W_gf, b_gf, W_a1, b_a1, W_a2, b_a2,
                      W_f1, b_f1, W_ff, b_ff)


# speedup vs baseline: 1.2483x; 1.2483x over previous
import jax
import jax.numpy as jnp
from jax import lax
from jax.experimental import pallas as pl
from jax.experimental.pallas import tpu as pltpu
from jax.experimental.pallas import tpu_sc as plsc

N = 50000
E = 800000
B = 512
EMB = 512
HID = 64

NC = 2
NS = 16
NW = NC * NS
CHUNK = 128
PER_W = 25088
E_PAD = PER_W * NW
NCHUNK = PER_W // CHUNK
NPAD = 50432
RPT = NPAD // NS
ZROWS = 394
YROWS = 50016
BLK = 2048
PBLK = BLK // 4
NPOOL = 51200
GRID = NPOOL // BLK
YR4 = YROWS // 4
NP4 = NPAD // 4
KB = 2
G = NCHUNK // KB
TOT_CHUNKS = E_PAD // CHUNK

_f32 = jnp.float32
_i32 = jnp.int32


def _deg_body(idx2_hbm, deg0_hbm, deg1_hbm, ones_v, idx_v, zbuf_v, acc_sh, sem_i):
    c = lax.axis_index("c")
    s = lax.axis_index("s")

    def _init(i, carry):
        ones_v[pl.ds(i * 16, 16)] = jnp.ones((16,), _f32)
        return carry
    lax.fori_loop(0, CHUNK // 16, _init, 0)

    def _zb(i, carry):
        zbuf_v[pl.ds(i * 16, 16)] = jnp.zeros((16,), _f32)
        return carry
    lax.fori_loop(0, RPT // 16, _zb, 0)
    pltpu.sync_copy(zbuf_v, acc_sh.at[pl.ds(s * RPT, RPT)])
    plsc.subcore_barrier()

    cbase = (s * NC + c) * NCHUNK

    def _start_idx(g, slot):
        pltpu.async_copy(idx2_hbm.at[pl.ds(cbase + g * KB, KB)],
                         idx_v.at[slot], sem_i.at[slot])

    def _wait_idx(slot):
        pltpu.make_async_copy(idx2_hbm.at[pl.ds(0, KB)],
                              idx_v.at[slot], sem_i.at[slot]).wait()

    _start_idx(0, 0)
    _start_idx(1, 1)
    _wait_idx(0)

    def _group(g, carry):
        slot = lax.rem(g, 2)
        nslot = lax.rem(g + 1, 2)
        for b in range(KB):
            pltpu.sync_copy(ones_v, acc_sh.at[idx_v.at[slot, b, 1]], add=True)

        @pl.when(g + 2 < G)
        def _pref():
            _start_idx(g + 2, slot)

        @pl.when(g + 1 < G)
        def _nw():
            _wait_idx(nslot)
        return carry
    lax.fori_loop(0, G, _group, 0)
    plsc.subcore_barrier()

    pltpu.sync_copy(acc_sh.at[pl.ds(s * RPT, RPT)], zbuf_v)

    @pl.when(c == 0)
    def _w0():
        pltpu.sync_copy(zbuf_v, deg0_hbm.at[pl.ds(s * RPT, RPT)])

    @pl.when(c == 1)
    def _w1():
        pltpu.sync_copy(zbuf_v, deg1_hbm.at[pl.ds(s * RPT, RPT)])


def _deg_pass(idx2):
    mesh = plsc.VectorSubcoreMesh(core_axis_name="c", subcore_axis_name="s")
    return pl.kernel(
        _deg_body,
        out_type=[jax.ShapeDtypeStruct((NPAD,), _f32),
                  jax.ShapeDtypeStruct((NPAD,), _f32)],
        mesh=mesh,
        scratch_types=[
            pltpu.VMEM((CHUNK,), _f32),
            pltpu.VMEM((2, KB, 2, CHUNK), _i32),
            pltpu.VMEM((RPT,), _f32),
            pltpu.VMEM_SHARED((NPAD,), _f32),
            pltpu.SemaphoreType.DMA((2,)),
        ],
        compiler_params=pltpu.CompilerParams(use_tc_tiling_on_sc=False),
    )(idx2)


def _conv_body(ylo_hbm, yhi_hbm, idx2_hbm, outlo_hbm, outhi_hbm,
               idx_v, rows_v, zbuf_v, acc_sh, sem_i, sem_g):
    c = lax.axis_index("c")
    s = lax.axis_index("s")

    def _zb(i, carry):
        r = i // 2
        zbuf_v[r, pl.ds((i % 2) * 16, 16)] = jnp.zeros((16,), _f32)
        return carry
    lax.fori_loop(0, ZROWS * 2, _zb, 0)

    def _zacc(i, carry):
        pltpu.sync_copy(zbuf_v, acc_sh.at[pl.ds(s * RPT + i * ZROWS, ZROWS)])
        return carry
    lax.fori_loop(0, RPT // ZROWS, _zacc, 0)
    plsc.subcore_barrier()

    cbase = (s * NC + c) * NCHUNK

    def _start_idx(g, slot):
        pltpu.async_copy(idx2_hbm.at[pl.ds(cbase + g * KB, KB)],
                         idx_v.at[slot], sem_i.at[slot])

    def _wait_idx(slot):
        pltpu.make_async_copy(idx2_hbm.at[pl.ds(0, KB)],
                              idx_v.at[slot], sem_i.at[slot]).wait()

    def _start_gathers(slot):
        for b in range(KB):
            @pl.when(c == 0)
            def _g0():
                pltpu.async_copy(ylo_hbm.at[idx_v.at[slot, b, 0]],
                                 rows_v.at[slot, b], sem_g.at[slot])

            @pl.when(c == 1)
            def _g1():
                pltpu.async_copy(yhi_hbm.at[idx_v.at[slot, b, 0]],
                                 rows_v.at[slot, b], sem_g.at[slot])

    def _wait_gathers(slot):
        for b in range(KB):
            pltpu.make_async_copy(ylo_hbm.at[pl.ds(0, CHUNK)],
                                  rows_v.at[slot, b], sem_g.at[slot]).wait()

    _start_idx(0, 0)
    _start_idx(1, 1)
    _wait_idx(0)
    _start_gathers(0)

    def _group(g, carry):
        slot = lax.rem(g, 2)
        nslot = lax.rem(g + 1, 2)
        _wait_gathers(slot)

        @pl.when(g + 1 < G)
        def _next():
            _wait_idx(nslot)
            _start_gathers(nslot)

        for b in range(KB):
            pltpu.sync_copy(rows_v.at[slot, b],
                            acc_sh.at[idx_v.at[slot, b, 1]], add=True)

        @pl.when(g + 2 < G)
        def _pref():
            _start_idx(g + 2, slot)
        return carry
    lax.fori_loop(0, G, _group, 0)
    plsc.subcore_barrier()

    def _wout(i, carry):
        r = s * RPT + i * ZROWS
        pltpu.sync_copy(acc_sh.at[pl.ds(r, ZROWS)], zbuf_v)

        @pl.when(c == 0)
        def _o0():
            pltpu.sync_copy(zbuf_v, outlo_hbm.at[pl.ds(r, ZROWS)])

        @pl.when(c == 1)
        def _o1():
            pltpu.sync_copy(zbuf_v, outhi_hbm.at[pl.ds(r, ZROWS)])
        return carry
    lax.fori_loop(0, RPT // ZROWS, _wout, 0)


def _conv_pass(ylo, yhi, idx2):
    mesh = plsc.VectorSubcoreMesh(core_axis_name="c", subcore_axis_name="s")
    return pl.kernel(
        _conv_body,
        out_type=[jax.ShapeDtypeStruct((NPAD, 32), _f32),
                  jax.ShapeDtypeStruct((NPAD, 32), _f32)],
        mesh=mesh,
        scratch_types=[
            pltpu.VMEM((2, KB, 2, CHUNK), _i32),
            pltpu.VMEM((2, KB, CHUNK, 32), _f32),
            pltpu.VMEM((ZROWS, 32), _f32),
            pltpu.VMEM_SHARED((NPAD, 32), _f32),
            pltpu.SemaphoreType.DMA((2,)),
            pltpu.SemaphoreType.DMA((2,)),
        ],
        compiler_params=pltpu.CompilerParams(use_tc_tiling_on_sc=False),
    )(ylo, yhi, idx2)


CPB = 128


def _idx_body(ei_ref, o_ref):
    i = pl.program_id(0)
    cols = i * (CPB * CHUNK) + lax.broadcasted_iota(_i32, (2, CPB * CHUNK), 1)
    v = jnp.where(cols < E, ei_ref[...], N)
    o_ref[...] = v.reshape(2, CPB, CHUNK).transpose(1, 0, 2)


def _idx_call(edge_index):
    return pl.pallas_call(
        _idx_body,
        grid=(TOT_CHUNKS // CPB,),
        in_specs=[pl.BlockSpec((2, CPB * CHUNK), lambda i: (0, i))],
        out_specs=pl.BlockSpec((CPB, 2, CHUNK), lambda i: (i, 0, 0)),
        out_shape=jax.ShapeDtypeStruct((TOT_CHUNKS, 2, CHUNK), _i32),
    )(edge_index)


def _rsqrt(v):
    r = lax.rsqrt(v)
    return r * (1.5 - 0.5 * v * r * r)


def _disp(d0_ref, d1_ref, d2_ref, d3_ref):
    cols = [jnp.broadcast_to(jnp.transpose(_rsqrt(d[...][0] + 1.0)), (PBLK, 32))
            for d in (d0_ref, d1_ref, d2_ref, d3_ref)]
    return jnp.concatenate(cols, axis=1)


_HIGH = lax.Precision.HIGHEST


def _y1_body(d0, d1, d2, d3, xp_ref, w4lo_ref, w4hi_ref, ylo_ref, yhi_ref):
    disp = _disp(d0, d1, d2, d3)
    xp = xp_ref[...]
    ylo_ref[...] = disp * jnp.dot(xp, w4lo_ref[...],
                                  preferred_element_type=_f32, precision=_HIGH)
    yhi_ref[...] = disp * jnp.dot(xp, w4hi_ref[...],
                                  preferred_element_type=_f32, precision=_HIGH)


def _y1_call(dk, x_p, W4lo, W4hi):
    dspec = pl.BlockSpec((1, 1, PBLK), lambda i: (i, 0, 0))
    row = lambda i: (i, 0)
    return pl.pallas_call(
        _y1_body,
        grid=(GRID,),
        in_specs=[dspec, dspec, dspec, dspec,
                  pl.BlockSpec((PBLK, 256), row),
                  pl.BlockSpec((256, 128), lambda i: (0, 0)),
                  pl.BlockSpec((256, 128), lambda i: (0, 0))],
        out_specs=[pl.BlockSpec((PBLK, 128), row),
                   pl.BlockSpec((PBLK, 128), row)],
        out_shape=[jax.ShapeDtypeStruct((YR4, 128), _f32),
                   jax.ShapeDtypeStruct((YR4, 128), _f32)],
    )(*dk, x_p, W4lo, W4hi)


def _y2_body(d0, d1, d2, d3, s1lo_ref, s1hi_ref, y1lo_ref, y1hi_ref,
             bplo_ref, bphi_ref, y2lo_ref, y2hi_ref):
    disp = _disp(d0, d1, d2, d3)
    glo = jnp.maximum(disp * (s1lo_ref[...] + y1lo_ref[...]) + bplo_ref[...], 0.0)
    ghi = jnp.maximum(disp * (s1hi_ref[...] + y1hi_ref[...]) + bphi_ref[...], 0.0)
    y2lo_ref[...] = disp * glo
    y2hi_ref[...] = disp * ghi


def _y2_call(dk, s1lo_p, s1hi_p, y1lo_p, y1hi_p, bplo, bphi):
    dspec = pl.BlockSpec((1, 1, PBLK), lambda i: (i, 0, 0))
    row = lambda i: (i, 0)
    return pl.pallas_call(
        _y2_body,
        grid=(GRID,),
        in_specs=[dspec, dspec, dspec, dspec,
                  pl.BlockSpec((PBLK, 128), row),
                  pl.BlockSpec((PBLK, 128), row),
                  pl.BlockSpec((PBLK, 128), row),
                  pl.BlockSpec((PBLK, 128), row),
                  pl.BlockSpec((1, 128), lambda i: (0, 0)),
                  pl.BlockSpec((1, 128), lambda i: (0, 0))],
        out_specs=[pl.BlockSpec((PBLK, 128), row),
                   pl.BlockSpec((PBLK, 128), row)],
        out_shape=[jax.ShapeDtypeStruct((YR4, 128), _f32),
                   jax.ShapeDtypeStruct((YR4, 128), _f32)],
    )(*dk, s1lo_p, s1hi_p, y1lo_p, y1hi_p, bplo, bphi)


def _emb_body(sm_ref, w1_ref, b1_ref, w2_ref, b2_ref, o_ref):
    h = jnp.maximum(
        jnp.dot(sm_ref[...][:, 0, :], w1_ref[...],
                preferred_element_type=_f32, precision=_HIGH)
        + b1_ref[...], 0.0)
    o_ref[...] = jnp.dot(h, w2_ref[...],
                         preferred_element_type=_f32, precision=_HIGH) + b2_ref[...]


def _emb_call(sm, W_e1, b_e1, W_e2, b_e2):
    return pl.pallas_call(
        _emb_body,
        out_shape=jax.ShapeDtypeStruct((B, HID), _f32),
    )(sm, W_e1, b_e1.reshape(1, -1), W_e2, b_e2.reshape(1, -1))


def _head_body(d0, d1, d2, d3, b0, b1, b2, b3,
               s2lo_ref, s2hi_ref, y2lo_ref, y2hi_ref, emb_ref,
               wc2_ref, bc2_ref, wgf_ref, bgf_ref,
               wa1_ref, ba1_ref, wa2_ref, ba2_ref, wf1_ref, bf1_ref,
               wff_ref, bff_ref, o_ref, acc_ref, cnt_ref):
    i = pl.program_id(0)

    @pl.when(i == 0)
    def _init():
        acc_ref[...] = jnp.zeros_like(acc_ref)
        cnt_ref[...] = jnp.zeros_like(cnt_ref)

    disp = _disp(d0, d1, d2, d3)
    colk = lax.broadcasted_iota(_i32, (PBLK, 128), 1) // 32
    node = i * BLK + 4 * lax.broadcasted_iota(_i32, (PBLK, 128), 0) + colk
    disp = jnp.where(node < N, disp, 0.0)
    t = jnp.concatenate([disp * (s2lo_ref[...] + y2lo_ref[...]),
                         disp * (s2hi_ref[...] + y2hi_ref[...])], axis=1)
    for k, bref in enumerate((b0, b1, b2, b3)):
        oh = (bref[...][0] ==
              lax.broadcasted_iota(_i32, (B, PBLK), 0)).astype(_f32)
        p = lax.dot_general(oh, t, (((1,), (0,)), ((), ())),
                            preferred_element_type=_f32, precision=_HIGH)
        acc_ref[:, :32] += p[:, 32 * k:32 * k + 32]
        acc_ref[:, 32:] += p[:, 128 + 32 * k:128 + 32 * k + 32]
        cnt_ref[...] += jnp.sum(oh, axis=1, keepdims=True)

    @pl.when(i == GRID - 1)
    def _final():
        cnt = cnt_ref[...]
        pooled64 = acc_ref[...] / jnp.maximum(cnt, 1.0)
        nz = (cnt > 0).astype(_f32)
        pooled = jnp.dot(pooled64, wc2_ref[...],
                         preferred_element_type=_f32,
                         precision=_HIGH) + bc2_ref[...] * nz
        gcn = jnp.dot(pooled, wgf_ref[...],
                      preferred_element_type=_f32, precision=_HIGH) + bgf_ref[...]
        emb = emb_ref[...]
        comb = jnp.concatenate([emb, gcn], axis=1)
        a = jnp.maximum(jnp.dot(comb, wa1_ref[...],
                                preferred_element_type=_f32,
                                precision=_HIGH) + ba1_ref[...], 0.0)
        att = jax.nn.sigmoid(jnp.dot(a, wa2_ref[...],
                                     preferred_element_type=_f32,
                                     precision=_HIGH) + ba2_ref[...])
        fused = emb * att + gcn * (1.0 - att)
        f1 = jnp.dot(fused, wf1_ref[...],
                     preferred_element_type=_f32, precision=_HIGH) + bf1_ref[...]
        o_ref[...] = jnp.dot(f1, wff_ref[...],
                             preferred_element_type=_f32,
                             precision=_HIGH) + bff_ref[...]


def _head_call(dk, bk, s2lo_p, s2hi_p, y2lo_p, y2hi_p, emb_out,
               W_c2, b_c2, W_gf, b_gf, W_a1, b_a1, W_a2, b_a2,
               W_f1, b_f1, W_ff, b_ff):
    dspec = pl.BlockSpec((1, 1, PBLK), lambda i: (i, 0, 0))
    row = lambda i: (i, 0)
    full = lambda i: (0, 0)
    return pl.pallas_call(
        _head_body,
        grid=(GRID,),
        in_specs=[dspec, dspec, dspec, dspec,
                  dspec, dspec, dspec, dspec,
                  pl.BlockSpec((PBLK, 128), row),
                  pl.BlockSpec((PBLK, 128), row),
                  pl.BlockSpec((PBLK, 128), row),
                  pl.BlockSpec((PBLK, 128), row),
                  pl.BlockSpec((B, HID), full),
                  pl.BlockSpec((HID, 128), full),
                  pl.BlockSpec((1, 128), full),
                  pl.BlockSpec((128, HID), full),
                  pl.BlockSpec((1, HID), full),
                  pl.BlockSpec((2 * HID, HID), full),
                  pl.BlockSpec((1, HID), full),
                  pl.BlockSpec((HID, 1), full),
                  pl.BlockSpec((1, 1), full),
                  pl.BlockSpec((HID, HID), full),
                  pl.BlockSpec((1, HID), full),
                  pl.BlockSpec((HID, 1), full),
                  pl.BlockSpec((1, 1), full)],
        out_specs=pl.BlockSpec((B, 1), full),
        out_shape=jax.ShapeDtypeStruct((B, 1), _f32),
        scratch_shapes=[pltpu.VMEM((B, HID), _f32),
                        pltpu.VMEM((B, 1), _f32)],
    )(*dk, *bk, s2lo_p, s2hi_p, y2lo_p, y2hi_p, emb_out,
      W_c2, b_c2.reshape(1, -1), W_gf, b_gf.reshape(1, -1),
      W_a1, b_a1.reshape(1, -1), W_a2, b_a2.reshape(1, -1),
      W_f1, b_f1.reshape(1, -1), W_ff, b_ff.reshape(1, -1))


def kernel(smiles_embedding, x, edge_index, batch,
           W_e1, b_e1, W_e2, b_e2, W_c1, b_c1, W_c2, b_c2, W_gf, b_gf,
           W_a1, b_a1, W_a2, b_a2, W_f1, b_f1, W_ff, b_ff):
    idx2 = _idx_call(edge_index.astype(_i32))

    deg0, deg1 = _deg_pass(idx2)
    degsp = jnp.concatenate([deg0 + deg1,
                             jnp.zeros((NPOOL - NPAD,), _f32)])
    dk = [degsp[k::4].reshape(GRID, 1, PBLK) for k in range(4)]
    batchp = jnp.pad(batch.astype(_i32), (0, NPOOL - N), constant_values=B)
    bk = [batchp[k::4].reshape(GRID, 1, PBLK) for k in range(4)]

    emb_out = _emb_call(smiles_embedding, W_e1, b_e1, W_e2, b_e2)

    x_p = x.reshape(N // 4, 256)
    eye4 = jnp.eye(4, dtype=_f32)
    W4lo = jnp.kron(eye4, W_c1[:, :32])
    W4hi = jnp.kron(eye4, W_c1[:, 32:])
    y1lo_p, y1hi_p = _y1_call(dk, x_p, W4lo, W4hi)
    s1lo, s1hi = _conv_pass(y1lo_p.reshape(YROWS, 32),
                            y1hi_p.reshape(YROWS, 32), idx2)

    bplo = jnp.tile(b_c1[:32], 4).reshape(1, 128)
    bphi = jnp.tile(b_c1[32:], 4).reshape(1, 128)
    y2lo_p, y2hi_p = _y2_call(dk, s1lo.reshape(NP4, 128),
                              s1hi.reshape(NP4, 128), y1lo_p, y1hi_p,
                              bplo, bphi)
    s2lo, s2hi = _conv_pass(y2lo_p.reshape(YROWS, 32),
                            y2hi_p.reshape(YROWS, 32), idx2)

    return _head_call(dk, bk, s2lo.reshape(NP4, 128), s2hi.reshape(NP4, 128),
                      y2lo_p, y2hi_p, emb_out,
                      W_c2, b_c2, W_gf, b_gf, W_a1, b_a1, W_a2, b_a2,
                      W_f1, b_f1, W_ff, b_ff)


# speedup vs baseline: 1.3799x; 1.1054x over previous
import jax
import jax.numpy as jnp
from jax import lax
from jax.experimental import pallas as pl
from jax.experimental.pallas import tpu as pltpu
from jax.experimental.pallas import tpu_sc as plsc

N = 50000
E = 800000
B = 512
EMB = 512
HID = 64

NC = 2
NS = 16
NW = NC * NS
CHUNK = 128
PER_W = 25088
E_PAD = PER_W * NW
NCHUNK = PER_W // CHUNK
NPAD = 50432
RPT = NPAD // NS
ZROWS = 394
YROWS = 50016
BLK = 2048
PBLK = BLK // 4
NPOOL = 51200
GRID = NPOOL // BLK
YR4 = YROWS // 4
NP4 = NPAD // 4
KB = 2
G = NCHUNK // KB
TOT_CHUNKS = E_PAD // CHUNK

_f32 = jnp.float32
_i32 = jnp.int32


def _deg_body(idx2_hbm, deg0_hbm, deg1_hbm, ones_v, idx_v, zbuf_v, acc_sh, sem_i):
    c = lax.axis_index("c")
    s = lax.axis_index("s")

    def _init(i, carry):
        ones_v[pl.ds(i * 16, 16)] = jnp.ones((16,), _f32)
        return carry
    lax.fori_loop(0, CHUNK // 16, _init, 0)

    def _zb(i, carry):
        zbuf_v[pl.ds(i * 16, 16)] = jnp.zeros((16,), _f32)
        return carry
    lax.fori_loop(0, RPT // 16, _zb, 0)
    pltpu.sync_copy(zbuf_v, acc_sh.at[pl.ds(s * RPT, RPT)])
    plsc.subcore_barrier()

    cbase = (s * NC + c) * NCHUNK

    def _start_idx(g, slot):
        pltpu.async_copy(idx2_hbm.at[pl.ds(cbase + g * KB, KB)],
                         idx_v.at[slot], sem_i.at[slot])

    def _wait_idx(slot):
        pltpu.make_async_copy(idx2_hbm.at[pl.ds(0, KB)],
                              idx_v.at[slot], sem_i.at[slot]).wait()

    _start_idx(0, 0)
    _start_idx(1, 1)
    _wait_idx(0)

    def _group(g, carry):
        slot = lax.rem(g, 2)
        nslot = lax.rem(g + 1, 2)
        for b in range(KB):
            pltpu.sync_copy(ones_v, acc_sh.at[idx_v.at[slot, b, 1]], add=True)

        @pl.when(g + 2 < G)
        def _pref():
            _start_idx(g + 2, slot)

        @pl.when(g + 1 < G)
        def _nw():
            _wait_idx(nslot)
        return carry
    lax.fori_loop(0, G, _group, 0)
    plsc.subcore_barrier()

    pltpu.sync_copy(acc_sh.at[pl.ds(s * RPT, RPT)], zbuf_v)

    @pl.when(c == 0)
    def _w0():
        pltpu.sync_copy(zbuf_v, deg0_hbm.at[pl.ds(s * RPT, RPT)])

    @pl.when(c == 1)
    def _w1():
        pltpu.sync_copy(zbuf_v, deg1_hbm.at[pl.ds(s * RPT, RPT)])


def _deg_pass(idx2):
    mesh = plsc.VectorSubcoreMesh(core_axis_name="c", subcore_axis_name="s")
    return pl.kernel(
        _deg_body,
        out_type=[jax.ShapeDtypeStruct((NPAD,), _f32),
                  jax.ShapeDtypeStruct((NPAD,), _f32)],
        mesh=mesh,
        scratch_types=[
            pltpu.VMEM((CHUNK,), _f32),
            pltpu.VMEM((2, KB, 2, CHUNK), _i32),
            pltpu.VMEM((RPT,), _f32),
            pltpu.VMEM_SHARED((NPAD,), _f32),
            pltpu.SemaphoreType.DMA((2,)),
        ],
        compiler_params=pltpu.CompilerParams(use_tc_tiling_on_sc=False),
    )(idx2)


def _conv_body(ylo_hbm, yhi_hbm, idx2_hbm, outlo_hbm, outhi_hbm,
               idx_v, rows_v, zbuf_v, acc_sh, sem_i, sem_g, sem_s):
    c = lax.axis_index("c")
    s = lax.axis_index("s")

    def _zb(i, carry):
        r = i // 2
        zbuf_v[r, pl.ds((i % 2) * 16, 16)] = jnp.zeros((16,), _f32)
        return carry
    lax.fori_loop(0, ZROWS * 2, _zb, 0)

    def _zacc(i, carry):
        pltpu.sync_copy(zbuf_v, acc_sh.at[pl.ds(s * RPT + i * ZROWS, ZROWS)])
        return carry
    lax.fori_loop(0, RPT // ZROWS, _zacc, 0)
    plsc.subcore_barrier()

    cbase = (s * NC + c) * NCHUNK

    def _start_idx(g, slot):
        pltpu.async_copy(idx2_hbm.at[pl.ds(cbase + g * KB, KB)],
                         idx_v.at[slot], sem_i.at[slot])

    def _wait_idx(slot):
        pltpu.make_async_copy(idx2_hbm.at[pl.ds(0, KB)],
                              idx_v.at[slot], sem_i.at[slot]).wait()

    def _start_gathers(slot):
        for b in range(KB):
            @pl.when(c == 0)
            def _g0():
                pltpu.async_copy(ylo_hbm.at[idx_v.at[slot, b, 0]],
                                 rows_v.at[slot, b], sem_g.at[slot])

            @pl.when(c == 1)
            def _g1():
                pltpu.async_copy(yhi_hbm.at[idx_v.at[slot, b, 0]],
                                 rows_v.at[slot, b], sem_g.at[slot])

    def _wait_gathers(slot):
        for b in range(KB):
            pltpu.make_async_copy(ylo_hbm.at[pl.ds(0, CHUNK)],
                                  rows_v.at[slot, b], sem_g.at[slot]).wait()

    def _wait_scatters(slot):
        for b in range(KB):
            pltpu.make_async_copy(ylo_hbm.at[pl.ds(0, CHUNK)],
                                  rows_v.at[slot, b], sem_s.at[slot]).wait()

    _start_idx(0, 0)
    _start_idx(1, 1)
    _wait_idx(0)
    _start_gathers(0)

    def _group(g, carry):
        slot = lax.rem(g, 2)
        nslot = lax.rem(g + 1, 2)
        _wait_gathers(slot)

        @pl.when(g + 1 < G)
        def _ni():
            _wait_idx(nslot)

        @pl.when(jnp.logical_and(g >= 1, g + 1 < G))
        def _ws():
            _wait_scatters(nslot)

        @pl.when(g + 1 < G)
        def _ng():
            _start_gathers(nslot)

        for b in range(KB):
            pltpu.async_copy(rows_v.at[slot, b],
                             acc_sh.at[idx_v.at[slot, b, 1]],
                             sem_s.at[slot], add=True)

        @pl.when(g + 2 < G)
        def _pref():
            _start_idx(g + 2, slot)
        return carry
    lax.fori_loop(0, G, _group, 0)
    _wait_scatters(0)
    _wait_scatters(1)
    plsc.subcore_barrier()

    def _wout(i, carry):
        r = s * RPT + i * ZROWS
        pltpu.sync_copy(acc_sh.at[pl.ds(r, ZROWS)], zbuf_v)

        @pl.when(c == 0)
        def _o0():
            pltpu.sync_copy(zbuf_v, outlo_hbm.at[pl.ds(r, ZROWS)])

        @pl.when(c == 1)
        def _o1():
            pltpu.sync_copy(zbuf_v, outhi_hbm.at[pl.ds(r, ZROWS)])
        return carry
    lax.fori_loop(0, RPT // ZROWS, _wout, 0)


def _conv_pass(ylo, yhi, idx2):
    mesh = plsc.VectorSubcoreMesh(core_axis_name="c", subcore_axis_name="s")
    return pl.kernel(
        _conv_body,
        out_type=[jax.ShapeDtypeStruct((NPAD, 32), _f32),
                  jax.ShapeDtypeStruct((NPAD, 32), _f32)],
        mesh=mesh,
        scratch_types=[
            pltpu.VMEM((2, KB, 2, CHUNK), _i32),
            pltpu.VMEM((2, KB, CHUNK, 32), _f32),
            pltpu.VMEM((ZROWS, 32), _f32),
            pltpu.VMEM_SHARED((NPAD, 32), _f32),
            pltpu.SemaphoreType.DMA((2,)),
            pltpu.SemaphoreType.DMA((2,)),
            pltpu.SemaphoreType.DMA((2,)),
        ],
        compiler_params=pltpu.CompilerParams(use_tc_tiling_on_sc=False),
    )(ylo, yhi, idx2)


CPB = 128


def _idx_body(ei_ref, o_ref):
    i = pl.program_id(0)
    cols = i * (CPB * CHUNK) + lax.broadcasted_iota(_i32, (2, CPB * CHUNK), 1)
    v = jnp.where(cols < E, ei_ref[...], N)
    o_ref[...] = v.reshape(2, CPB, CHUNK).transpose(1, 0, 2)


def _idx_call(edge_index):
    return pl.pallas_call(
        _idx_body,
        grid=(TOT_CHUNKS // CPB,),
        in_specs=[pl.BlockSpec((2, CPB * CHUNK), lambda i: (0, i))],
        out_specs=pl.BlockSpec((CPB, 2, CHUNK), lambda i: (i, 0, 0)),
        out_shape=jax.ShapeDtypeStruct((TOT_CHUNKS, 2, CHUNK), _i32),
    )(edge_index)


def _rsqrt(v):
    r = lax.rsqrt(v)
    return r * (1.5 - 0.5 * v * r * r)


def _disp(d0_ref, d1_ref, d2_ref, d3_ref):
    cols = [jnp.broadcast_to(jnp.transpose(_rsqrt(d[...][0] + 1.0)), (PBLK, 32))
            for d in (d0_ref, d1_ref, d2_ref, d3_ref)]
    return jnp.concatenate(cols, axis=1)


_HIGH = lax.Precision.HIGHEST


def _y1_body(d0, d1, d2, d3, xp_ref, w4lo_ref, w4hi_ref, ylo_ref, yhi_ref):
    disp = _disp(d0, d1, d2, d3)
    xp = xp_ref[...]
    ylo_ref[...] = disp * jnp.dot(xp, w4lo_ref[...],
                                  preferred_element_type=_f32, precision=_HIGH)
    yhi_ref[...] = disp * jnp.dot(xp, w4hi_ref[...],
                                  preferred_element_type=_f32, precision=_HIGH)


def _y1_call(dk, x_p, W4lo, W4hi):
    dspec = pl.BlockSpec((1, 1, PBLK), lambda i: (i, 0, 0))
    row = lambda i: (i, 0)
    return pl.pallas_call(
        _y1_body,
        grid=(GRID,),
        in_specs=[dspec, dspec, dspec, dspec,
                  pl.BlockSpec((PBLK, 256), row),
                  pl.BlockSpec((256, 128), lambda i: (0, 0)),
                  pl.BlockSpec((256, 128), lambda i: (0, 0))],
        out_specs=[pl.BlockSpec((PBLK, 128), row),
                   pl.BlockSpec((PBLK, 128), row)],
        out_shape=[jax.ShapeDtypeStruct((YR4, 128), _f32),
                   jax.ShapeDtypeStruct((YR4, 128), _f32)],
    )(*dk, x_p, W4lo, W4hi)


def _y2_body(d0, d1, d2, d3, s1lo_ref, s1hi_ref, y1lo_ref, y1hi_ref,
             bplo_ref, bphi_ref, y2lo_ref, y2hi_ref):
    disp = _disp(d0, d1, d2, d3)
    glo = jnp.maximum(disp * (s1lo_ref[...] + y1lo_ref[...]) + bplo_ref[...], 0.0)
    ghi = jnp.maximum(disp * (s1hi_ref[...] + y1hi_ref[...]) + bphi_ref[...], 0.0)
    y2lo_ref[...] = disp * glo
    y2hi_ref[...] = disp * ghi


def _y2_call(dk, s1lo_p, s1hi_p, y1lo_p, y1hi_p, bplo, bphi):
    dspec = pl.BlockSpec((1, 1, PBLK), lambda i: (i, 0, 0))
    row = lambda i: (i, 0)
    return pl.pallas_call(
        _y2_body,
        grid=(GRID,),
        in_specs=[dspec, dspec, dspec, dspec,
                  pl.BlockSpec((PBLK, 128), row),
                  pl.BlockSpec((PBLK, 128), row),
                  pl.BlockSpec((PBLK, 128), row),
                  pl.BlockSpec((PBLK, 128), row),
                  pl.BlockSpec((1, 128), lambda i: (0, 0)),
                  pl.BlockSpec((1, 128), lambda i: (0, 0))],
        out_specs=[pl.BlockSpec((PBLK, 128), row),
                   pl.BlockSpec((PBLK, 128), row)],
        out_shape=[jax.ShapeDtypeStruct((YR4, 128), _f32),
                   jax.ShapeDtypeStruct((YR4, 128), _f32)],
    )(*dk, s1lo_p, s1hi_p, y1lo_p, y1hi_p, bplo, bphi)


def _emb_body(sm_ref, w1_ref, b1_ref, w2_ref, b2_ref, o_ref):
    h = jnp.maximum(
        jnp.dot(sm_ref[...][:, 0, :], w1_ref[...],
                preferred_element_type=_f32, precision=_HIGH)
        + b1_ref[...], 0.0)
    o_ref[...] = jnp.dot(h, w2_ref[...],
                         preferred_element_type=_f32, precision=_HIGH) + b2_ref[...]


def _emb_call(sm, W_e1, b_e1, W_e2, b_e2):
    return pl.pallas_call(
        _emb_body,
        out_shape=jax.ShapeDtypeStruct((B, HID), _f32),
    )(sm, W_e1, b_e1.reshape(1, -1), W_e2, b_e2.reshape(1, -1))


def _head_body(d0, d1, d2, d3, b0, b1, b2, b3,
               s2lo_ref, s2hi_ref, y2lo_ref, y2hi_ref, emb_ref,
               wc2_ref, bc2_ref, wgf_ref, bgf_ref,
               wa1_ref, ba1_ref, wa2_ref, ba2_ref, wf1_ref, bf1_ref,
               wff_ref, bff_ref, o_ref, acc_ref, cnt_ref):
    i = pl.program_id(0)

    @pl.when(i == 0)
    def _init():
        acc_ref[...] = jnp.zeros_like(acc_ref)
        cnt_ref[...] = jnp.zeros_like(cnt_ref)

    disp = _disp(d0, d1, d2, d3)
    colk = lax.broadcasted_iota(_i32, (PBLK, 128), 1) // 32
    node = i * BLK + 4 * lax.broadcasted_iota(_i32, (PBLK, 128), 0) + colk
    disp = jnp.where(node < N, disp, 0.0)
    t = jnp.concatenate([disp * (s2lo_ref[...] + y2lo_ref[...]),
                         disp * (s2hi_ref[...] + y2hi_ref[...])], axis=1)
    t_hi = t.astype(jnp.bfloat16)
    t_lo = (t - t_hi.astype(_f32)).astype(jnp.bfloat16)
    t2 = jnp.concatenate([t_hi, t_lo], axis=1)
    for k, bref in enumerate((b0, b1, b2, b3)):
        oh = (bref[...][0] ==
              lax.broadcasted_iota(_i32, (B, PBLK), 0)).astype(jnp.bfloat16)
        p2 = lax.dot_general(oh, t2, (((1,), (0,)), ((), ())),
                             preferred_element_type=_f32)
        p = p2[:, :256] + p2[:, 256:]
        acc_ref[:, :32] += p[:, 32 * k:32 * k + 32]
        acc_ref[:, 32:] += p[:, 128 + 32 * k:128 + 32 * k + 32]
        cnt_ref[...] += jnp.sum(oh, axis=1, keepdims=True)

    @pl.when(i == GRID - 1)
    def _final():
        cnt = cnt_ref[...]
        pooled64 = acc_ref[...] / jnp.maximum(cnt, 1.0)
        nz = (cnt > 0).astype(_f32)
        pooled = jnp.dot(pooled64, wc2_ref[...],
                         preferred_element_type=_f32,
                         precision=_HIGH) + bc2_ref[...] * nz
        gcn = jnp.dot(pooled, wgf_ref[...],
                      preferred_element_type=_f32, precision=_HIGH) + bgf_ref[...]
        emb = emb_ref[...]
        comb = jnp.concatenate([emb, gcn], axis=1)
        a = jnp.maximum(jnp.dot(comb, wa1_ref[...],
                                preferred_element_type=_f32,
                                precision=_HIGH) + ba1_ref[...], 0.0)
        att = jax.nn.sigmoid(jnp.dot(a, wa2_ref[...],
                                     preferred_element_type=_f32,
                                     precision=_HIGH) + ba2_ref[...])
        fused = emb * att + gcn * (1.0 - att)
        f1 = jnp.dot(fused, wf1_ref[...],
                     preferred_element_type=_f32, precision=_HIGH) + bf1_ref[...]
        o_ref[...] = jnp.dot(f1, wff_ref[...],
                             preferred_element_type=_f32,
                             precision=_HIGH) + bff_ref[...]


def _head_call(dk, bk, s2lo_p, s2hi_p, y2lo_p, y2hi_p, emb_out,
               W_c2, b_c2, W_gf, b_gf, W_a1, b_a1, W_a2, b_a2,
               W_f1, b_f1, W_ff, b_ff):
    dspec = pl.BlockSpec((1, 1, PBLK), lambda i: (i, 0, 0))
    row = lambda i: (i, 0)
    full = lambda i: (0, 0)
    return pl.pallas_call(
        _head_body,
        grid=(GRID,),
        in_specs=[dspec, dspec, dspec, dspec,
                  dspec, dspec, dspec, dspec,
                  pl.BlockSpec((PBLK, 128), row),
                  pl.BlockSpec((PBLK, 128), row),
                  pl.BlockSpec((PBLK, 128), row),
                  pl.BlockSpec((PBLK, 128), row),
                  pl.BlockSpec((B, HID), full),
                  pl.BlockSpec((HID, 128), full),
                  pl.BlockSpec((1, 128), full),
                  pl.BlockSpec((128, HID), full),
                  pl.BlockSpec((1, HID), full),
                  pl.BlockSpec((2 * HID, HID), full),
                  pl.BlockSpec((1, HID), full),
                  pl.BlockSpec((HID, 1), full),
                  pl.BlockSpec((1, 1), full),
                  pl.BlockSpec((HID, HID), full),
                  pl.BlockSpec((1, HID), full),
                  pl.BlockSpec((HID, 1), full),
                  pl.BlockSpec((1, 1), full)],
        out_specs=pl.BlockSpec((B, 1), full),
        out_shape=jax.ShapeDtypeStruct((B, 1), _f32),
        scratch_shapes=[pltpu.VMEM((B, HID), _f32),
                        pltpu.VMEM((B, 1), _f32)],
    )(*dk, *bk, s2lo_p, s2hi_p, y2lo_p, y2hi_p, emb_out,
      W_c2, b_c2.reshape(1, -1), W_gf, b_gf.reshape(1, -1),
      W_a1, b_a1.reshape(1, -1), W_a2, b_a2.reshape(1, -1),
      W_f1, b_f1.reshape(1, -1), W_ff, b_ff.reshape(1, -1))


def kernel(smiles_embedding, x, edge_index, batch,
           W_e1, b_e1, W_e2, b_e2, W_c1, b_c1, W_c2, b_c2, W_gf, b_gf,
           W_a1, b_a1, W_a2, b_a2, W_f1, b_f1, W_ff, b_ff):
    idx2 = _idx_call(edge_index.astype(_i32))

    deg0, deg1 = _deg_pass(idx2)
    degsp = jnp.concatenate([deg0 + deg1,
                             jnp.zeros((NPOOL - NPAD,), _f32)])
    dk = [degsp[k::4].reshape(GRID, 1, PBLK) for k in range(4)]
    batchp = jnp.pad(batch.astype(_i32), (0, NPOOL - N), constant_values=B)
    bk = [batchp[k::4].reshape(GRID, 1, PBLK) for k in range(4)]

    emb_out = _emb_call(smiles_embedding, W_e1, b_e1, W_e2, b_e2)

    x_p = x.reshape(N // 4, 256)
    eye4 = jnp.eye(4, dtype=_f32)
    W4lo = jnp.kron(eye4, W_c1[:, :32])
    W4hi = jnp.kron(eye4, W_c1[:, 32:])
    y1lo_p, y1hi_p = _y1_call(dk, x_p, W4lo, W4hi)
    s1lo, s1hi = _conv_pass(y1lo_p.reshape(YROWS, 32),
                            y1hi_p.reshape(YROWS, 32), idx2)

    bplo = jnp.tile(b_c1[:32], 4).reshape(1, 128)
    bphi = jnp.tile(b_c1[32:], 4).reshape(1, 128)
    y2lo_p, y2hi_p = _y2_call(dk, s1lo.reshape(NP4, 128),
                              s1hi.reshape(NP4, 128), y1lo_p, y1hi_p,
                              bplo, bphi)
    s2lo, s2hi = _conv_pass(y2lo_p.reshape(YROWS, 32),
                            y2hi_p.reshape(YROWS, 32), idx2)

    return _head_call(dk, bk, s2lo.reshape(NP4, 128), s2hi.reshape(NP4, 128),
                      y2lo_p, y2hi_p, emb_out,
                      W_c2, b_c2, W_gf, b_gf, W_a1, b_a1, W_a2, b_a2,
                      W_f1, b_f1, W_ff, b_ff)


# speedup vs baseline: 1.3832x; 1.0024x over previous
import jax
import jax.numpy as jnp
from jax import lax
from jax.experimental import pallas as pl
from jax.experimental.pallas import tpu as pltpu
from jax.experimental.pallas import tpu_sc as plsc

N = 50000
E = 800000
B = 512
EMB = 512
HID = 64

NC = 2
NS = 16
NW = NC * NS
CHUNK = 128
PER_W = 25088
E_PAD = PER_W * NW
NCHUNK = PER_W // CHUNK
NPAD = 50432
RPT = NPAD // NS
ZROWS = 394
YROWS = 50016
BLK = 2048
PBLK = BLK // 4
NPOOL = 51200
GRID = NPOOL // BLK
YR4 = YROWS // 4
NP4 = NPAD // 4
KB = 2
G = NCHUNK // KB
TOT_CHUNKS = E_PAD // CHUNK

_f32 = jnp.float32
_i32 = jnp.int32


def _deg_body(idx2_hbm, deg0_hbm, deg1_hbm, ones_v, idx_v, zbuf_v, acc_sh, sem_i):
    c = lax.axis_index("c")
    s = lax.axis_index("s")

    def _init(i, carry):
        ones_v[pl.ds(i * 16, 16)] = jnp.ones((16,), _f32)
        return carry
    lax.fori_loop(0, CHUNK // 16, _init, 0)

    def _zb(i, carry):
        zbuf_v[pl.ds(i * 16, 16)] = jnp.zeros((16,), _f32)
        return carry
    lax.fori_loop(0, RPT // 16, _zb, 0)
    pltpu.sync_copy(zbuf_v, acc_sh.at[pl.ds(s * RPT, RPT)])
    plsc.subcore_barrier()

    cbase = (s * NC + c) * NCHUNK

    def _start_idx(g, slot):
        pltpu.async_copy(idx2_hbm.at[pl.ds(cbase + g * KB, KB)],
                         idx_v.at[slot], sem_i.at[slot])

    def _wait_idx(slot):
        pltpu.make_async_copy(idx2_hbm.at[pl.ds(0, KB)],
                              idx_v.at[slot], sem_i.at[slot]).wait()

    _start_idx(0, 0)
    _start_idx(1, 1)
    _wait_idx(0)

    def _group(g, carry):
        slot = lax.rem(g, 2)
        nslot = lax.rem(g + 1, 2)
        for b in range(KB):
            pltpu.sync_copy(ones_v, acc_sh.at[idx_v.at[slot, b, 1]], add=True)

        @pl.when(g + 2 < G)
        def _pref():
            _start_idx(g + 2, slot)

        @pl.when(g + 1 < G)
        def _nw():
            _wait_idx(nslot)
        return carry
    lax.fori_loop(0, G, _group, 0)
    plsc.subcore_barrier()

    pltpu.sync_copy(acc_sh.at[pl.ds(s * RPT, RPT)], zbuf_v)

    @pl.when(c == 0)
    def _w0():
        pltpu.sync_copy(zbuf_v, deg0_hbm.at[pl.ds(s * RPT, RPT)])

    @pl.when(c == 1)
    def _w1():
        pltpu.sync_copy(zbuf_v, deg1_hbm.at[pl.ds(s * RPT, RPT)])


def _deg_pass(idx2):
    mesh = plsc.VectorSubcoreMesh(core_axis_name="c", subcore_axis_name="s")
    return pl.kernel(
        _deg_body,
        out_type=[jax.ShapeDtypeStruct((NPAD,), _f32),
                  jax.ShapeDtypeStruct((NPAD,), _f32)],
        mesh=mesh,
        scratch_types=[
            pltpu.VMEM((CHUNK,), _f32),
            pltpu.VMEM((2, KB, 2, CHUNK), _i32),
            pltpu.VMEM((RPT,), _f32),
            pltpu.VMEM_SHARED((NPAD,), _f32),
            pltpu.SemaphoreType.DMA((2,)),
        ],
        compiler_params=pltpu.CompilerParams(use_tc_tiling_on_sc=False),
    )(idx2)


def _conv_body(ylo_hbm, yhi_hbm, idx2_hbm, outlo_hbm, outhi_hbm,
               idx_v, rows_v, zbuf_v, acc_sh, sem_i, sem_g, sem_s):
    c = lax.axis_index("c")
    s = lax.axis_index("s")
    cbase = (s * NC + c) * NCHUNK

    def _start_idx(g, slot):
        pltpu.async_copy(idx2_hbm.at[pl.ds(cbase + g * KB, KB)],
                         idx_v.at[slot], sem_i.at[slot])

    _start_idx(0, 0)
    _start_idx(1, 1)

    def _zb(i, carry):
        r = i // 2
        zbuf_v[r, pl.ds((i % 2) * 16, 16)] = jnp.zeros((16,), _f32)
        return carry
    lax.fori_loop(0, ZROWS * 2, _zb, 0)

    def _zacc(i, carry):
        pltpu.sync_copy(zbuf_v, acc_sh.at[pl.ds(s * RPT + i * ZROWS, ZROWS)])
        return carry
    lax.fori_loop(0, RPT // ZROWS, _zacc, 0)

    def _wait_idx(slot):
        pltpu.make_async_copy(idx2_hbm.at[pl.ds(0, KB)],
                              idx_v.at[slot], sem_i.at[slot]).wait()

    def _start_gathers(slot):
        for b in range(KB):
            @pl.when(c == 0)
            def _g0():
                pltpu.async_copy(ylo_hbm.at[idx_v.at[slot, b, 0]],
                                 rows_v.at[slot, b], sem_g.at[slot])

            @pl.when(c == 1)
            def _g1():
                pltpu.async_copy(yhi_hbm.at[idx_v.at[slot, b, 0]],
                                 rows_v.at[slot, b], sem_g.at[slot])

    def _wait_gathers(slot):
        for b in range(KB):
            pltpu.make_async_copy(ylo_hbm.at[pl.ds(0, CHUNK)],
                                  rows_v.at[slot, b], sem_g.at[slot]).wait()

    def _wait_scatters(slot):
        for b in range(KB):
            pltpu.make_async_copy(ylo_hbm.at[pl.ds(0, CHUNK)],
                                  rows_v.at[slot, b], sem_s.at[slot]).wait()

    _wait_idx(0)
    _start_gathers(0)
    plsc.subcore_barrier()

    def _group(g, carry):
        slot = lax.rem(g, 2)
        nslot = lax.rem(g + 1, 2)
        _wait_gathers(slot)

        @pl.when(g + 1 < G)
        def _ni():
            _wait_idx(nslot)

        @pl.when(jnp.logical_and(g >= 1, g + 1 < G))
        def _ws():
            _wait_scatters(nslot)

        @pl.when(g + 1 < G)
        def _ng():
            _start_gathers(nslot)

        for b in range(KB):
            pltpu.async_copy(rows_v.at[slot, b],
                             acc_sh.at[idx_v.at[slot, b, 1]],
                             sem_s.at[slot], add=True)

        @pl.when(g + 2 < G)
        def _pref():
            _start_idx(g + 2, slot)
        return carry
    lax.fori_loop(0, G, _group, 0)
    _wait_scatters(0)
    _wait_scatters(1)
    plsc.subcore_barrier()

    def _wout(i, carry):
        r = s * RPT + i * ZROWS
        pltpu.sync_copy(acc_sh.at[pl.ds(r, ZROWS)], zbuf_v)

        @pl.when(c == 0)
        def _o0():
            pltpu.sync_copy(zbuf_v, outlo_hbm.at[pl.ds(r, ZROWS)])

        @pl.when(c == 1)
        def _o1():
            pltpu.sync_copy(zbuf_v, outhi_hbm.at[pl.ds(r, ZROWS)])
        return carry
    lax.fori_loop(0, RPT // ZROWS, _wout, 0)


def _conv_pass(ylo, yhi, idx2):
    mesh = plsc.VectorSubcoreMesh(core_axis_name="c", subcore_axis_name="s")
    return pl.kernel(
        _conv_body,
        out_type=[jax.ShapeDtypeStruct((NPAD, 32), _f32),
                  jax.ShapeDtypeStruct((NPAD, 32), _f32)],
        mesh=mesh,
        scratch_types=[
            pltpu.VMEM((2, KB, 2, CHUNK), _i32),
            pltpu.VMEM((2, KB, CHUNK, 32), _f32),
            pltpu.VMEM((ZROWS, 32), _f32),
            pltpu.VMEM_SHARED((NPAD, 32), _f32),
            pltpu.SemaphoreType.DMA((2,)),
            pltpu.SemaphoreType.DMA((2,)),
            pltpu.SemaphoreType.DMA((2,)),
        ],
        compiler_params=pltpu.CompilerParams(use_tc_tiling_on_sc=False),
    )(ylo, yhi, idx2)


CPB = 128


def _idx_body(ei_ref, o_ref):
    i = pl.program_id(0)
    cols = i * (CPB * CHUNK) + lax.broadcasted_iota(_i32, (2, CPB * CHUNK), 1)
    v = jnp.where(cols < E, ei_ref[...], N)
    o_ref[...] = v.reshape(2, CPB, CHUNK).transpose(1, 0, 2)


def _idx_call(edge_index):
    return pl.pallas_call(
        _idx_body,
        grid=(TOT_CHUNKS // CPB,),
        in_specs=[pl.BlockSpec((2, CPB * CHUNK), lambda i: (0, i))],
        out_specs=pl.BlockSpec((CPB, 2, CHUNK), lambda i: (i, 0, 0)),
        out_shape=jax.ShapeDtypeStruct((TOT_CHUNKS, 2, CHUNK), _i32),
    )(edge_index)


def _rsqrt(v):
    r = lax.rsqrt(v)
    return r * (1.5 - 0.5 * v * r * r)


def _disp(d0_ref, d1_ref, d2_ref, d3_ref):
    cols = [jnp.broadcast_to(jnp.transpose(_rsqrt(d[...][0] + 1.0)), (PBLK, 32))
            for d in (d0_ref, d1_ref, d2_ref, d3_ref)]
    return jnp.concatenate(cols, axis=1)


_HIGH = lax.Precision.HIGHEST


def _y1_body(d0, d1, d2, d3, xp_ref, w4lo_ref, w4hi_ref, ylo_ref, yhi_ref):
    disp = _disp(d0, d1, d2, d3)
    xp = xp_ref[...]
    ylo_ref[...] = disp * jnp.dot(xp, w4lo_ref[...],
                                  preferred_element_type=_f32, precision=_HIGH)
    yhi_ref[...] = disp * jnp.dot(xp, w4hi_ref[...],
                                  preferred_element_type=_f32, precision=_HIGH)


def _y1_call(dk, x_p, W4lo, W4hi):
    dspec = pl.BlockSpec((1, 1, PBLK), lambda i: (i, 0, 0))
    row = lambda i: (i, 0)
    return pl.pallas_call(
        _y1_body,
        grid=(GRID,),
        in_specs=[dspec, dspec, dspec, dspec,
                  pl.BlockSpec((PBLK, 256), row),
                  pl.BlockSpec((256, 128), lambda i: (0, 0)),
                  pl.BlockSpec((256, 128), lambda i: (0, 0))],
        out_specs=[pl.BlockSpec((PBLK, 128), row),
                   pl.BlockSpec((PBLK, 128), row)],
        out_shape=[jax.ShapeDtypeStruct((YR4, 128), _f32),
                   jax.ShapeDtypeStruct((YR4, 128), _f32)],
    )(*dk, x_p, W4lo, W4hi)


def _y2_body(d0, d1, d2, d3, s1lo_ref, s1hi_ref, y1lo_ref, y1hi_ref,
             bplo_ref, bphi_ref, y2lo_ref, y2hi_ref):
    disp = _disp(d0, d1, d2, d3)
    glo = jnp.maximum(disp * (s1lo_ref[...] + y1lo_ref[...]) + bplo_ref[...], 0.0)
    ghi = jnp.maximum(disp * (s1hi_ref[...] + y1hi_ref[...]) + bphi_ref[...], 0.0)
    y2lo_ref[...] = disp * glo
    y2hi_ref[...] = disp * ghi


def _y2_call(dk, s1lo_p, s1hi_p, y1lo_p, y1hi_p, bplo, bphi):
    dspec = pl.BlockSpec((1, 1, PBLK), lambda i: (i, 0, 0))
    row = lambda i: (i, 0)
    return pl.pallas_call(
        _y2_body,
        grid=(GRID,),
        in_specs=[dspec, dspec, dspec, dspec,
                  pl.BlockSpec((PBLK, 128), row),
                  pl.BlockSpec((PBLK, 128), row),
                  pl.BlockSpec((PBLK, 128), row),
                  pl.BlockSpec((PBLK, 128), row),
                  pl.BlockSpec((1, 128), lambda i: (0, 0)),
                  pl.BlockSpec((1, 128), lambda i: (0, 0))],
        out_specs=[pl.BlockSpec((PBLK, 128), row),
                   pl.BlockSpec((PBLK, 128), row)],
        out_shape=[jax.ShapeDtypeStruct((YR4, 128), _f32),
                   jax.ShapeDtypeStruct((YR4, 128), _f32)],
    )(*dk, s1lo_p, s1hi_p, y1lo_p, y1hi_p, bplo, bphi)


def _emb_body(sm_ref, w1_ref, b1_ref, w2_ref, b2_ref, o_ref):
    h = jnp.maximum(
        jnp.dot(sm_ref[...][:, 0, :], w1_ref[...],
                preferred_element_type=_f32, precision=_HIGH)
        + b1_ref[...], 0.0)
    o_ref[...] = jnp.dot(h, w2_ref[...],
                         preferred_element_type=_f32, precision=_HIGH) + b2_ref[...]


def _emb_call(sm, W_e1, b_e1, W_e2, b_e2):
    return pl.pallas_call(
        _emb_body,
        out_shape=jax.ShapeDtypeStruct((B, HID), _f32),
    )(sm, W_e1, b_e1.reshape(1, -1), W_e2, b_e2.reshape(1, -1))


def _head_body(d0, d1, d2, d3, b0, b1, b2, b3,
               s2lo_ref, s2hi_ref, y2lo_ref, y2hi_ref, emb_ref,
               wc2_ref, bc2_ref, wgf_ref, bgf_ref,
               wa1_ref, ba1_ref, wa2_ref, ba2_ref, wf1_ref, bf1_ref,
               wff_ref, bff_ref, o_ref, acc_ref, cnt_ref):
    i = pl.program_id(0)

    @pl.when(i == 0)
    def _init():
        acc_ref[...] = jnp.zeros_like(acc_ref)
        cnt_ref[...] = jnp.zeros_like(cnt_ref)

    disp = _disp(d0, d1, d2, d3)
    colk = lax.broadcasted_iota(_i32, (PBLK, 128), 1) // 32
    node = i * BLK + 4 * lax.broadcasted_iota(_i32, (PBLK, 128), 0) + colk
    disp = jnp.where(node < N, disp, 0.0)
    t = jnp.concatenate([disp * (s2lo_ref[...] + y2lo_ref[...]),
                         disp * (s2hi_ref[...] + y2hi_ref[...])], axis=1)
    t_hi = t.astype(jnp.bfloat16)
    t_lo = (t - t_hi.astype(_f32)).astype(jnp.bfloat16)
    t2 = jnp.concatenate([t_hi, t_lo], axis=1)
    for k, bref in enumerate((b0, b1, b2, b3)):
        oh = (bref[...][0] ==
              lax.broadcasted_iota(_i32, (B, PBLK), 0)).astype(jnp.bfloat16)
        p2 = lax.dot_general(oh, t2, (((1,), (0,)), ((), ())),
                             preferred_element_type=_f32)
        p = p2[:, :256] + p2[:, 256:]
        acc_ref[:, :32] += p[:, 32 * k:32 * k + 32]
        acc_ref[:, 32:] += p[:, 128 + 32 * k:128 + 32 * k + 32]
        cnt_ref[...] += jnp.sum(oh, axis=1, keepdims=True)

    @pl.when(i == GRID - 1)
    def _final():
        cnt = cnt_ref[...]
        pooled64 = acc_ref[...] / jnp.maximum(cnt, 1.0)
        nz = (cnt > 0).astype(_f32)
        pooled = jnp.dot(pooled64, wc2_ref[...],
                         preferred_element_type=_f32,
                         precision=_HIGH) + bc2_ref[...] * nz
        gcn = jnp.dot(pooled, wgf_ref[...],
                      preferred_element_type=_f32, precision=_HIGH) + bgf_ref[...]
        emb = emb_ref[...]
        comb = jnp.concatenate([emb, gcn], axis=1)
        a = jnp.maximum(jnp.dot(comb, wa1_ref[...],
                                preferred_element_type=_f32,
                                precision=_HIGH) + ba1_ref[...], 0.0)
        att = jax.nn.sigmoid(jnp.dot(a, wa2_ref[...],
                                     preferred_element_type=_f32,
                                     precision=_HIGH) + ba2_ref[...])
        fused = emb * att + gcn * (1.0 - att)
        f1 = jnp.dot(fused, wf1_ref[...],
                     preferred_element_type=_f32, precision=_HIGH) + bf1_ref[...]
        o_ref[...] = jnp.dot(f1, wff_ref[...],
                             preferred_element_type=_f32,
                             precision=_HIGH) + bff_ref[...]


def _head_call(dk, bk, s2lo_p, s2hi_p, y2lo_p, y2hi_p, emb_out,
               W_c2, b_c2, W_gf, b_gf, W_a1, b_a1, W_a2, b_a2,
               W_f1, b_f1, W_ff, b_ff):
    dspec = pl.BlockSpec((1, 1, PBLK), lambda i: (i, 0, 0))
    row = lambda i: (i, 0)
    full = lambda i: (0, 0)
    return pl.pallas_call(
        _head_body,
        grid=(GRID,),
        in_specs=[dspec, dspec, dspec, dspec,
                  dspec, dspec, dspec, dspec,
                  pl.BlockSpec((PBLK, 128), row),
                  pl.BlockSpec((PBLK, 128), row),
                  pl.BlockSpec((PBLK, 128), row),
                  pl.BlockSpec((PBLK, 128), row),
                  pl.BlockSpec((B, HID), full),
                  pl.BlockSpec((HID, 128), full),
                  pl.BlockSpec((1, 128), full),
                  pl.BlockSpec((128, HID), full),
                  pl.BlockSpec((1, HID), full),
                  pl.BlockSpec((2 * HID, HID), full),
                  pl.BlockSpec((1, HID), full),
                  pl.BlockSpec((HID, 1), full),
                  pl.BlockSpec((1, 1), full),
                  pl.BlockSpec((HID, HID), full),
                  pl.BlockSpec((1, HID), full),
                  pl.BlockSpec((HID, 1), full),
                  pl.BlockSpec((1, 1), full)],
        out_specs=pl.BlockSpec((B, 1), full),
        out_shape=jax.ShapeDtypeStruct((B, 1), _f32),
        scratch_shapes=[pltpu.VMEM((B, HID), _f32),
                        pltpu.VMEM((B, 1), _f32)],
    )(*dk, *bk, s2lo_p, s2hi_p, y2lo_p, y2hi_p, emb_out,
      W_c2, b_c2.reshape(1, -1), W_gf, b_gf.reshape(1, -1),
      W_a1, b_a1.reshape(1, -1), W_a2, b_a2.reshape(1, -1),
      W_f1, b_f1.reshape(1, -1), W_ff, b_ff.reshape(1, -1))


def kernel(smiles_embedding, x, edge_index, batch,
           W_e1, b_e1, W_e2, b_e2, W_c1, b_c1, W_c2, b_c2, W_gf, b_gf,
           W_a1, b_a1, W_a2, b_a2, W_f1, b_f1, W_ff, b_ff):
    idx2 = _idx_call(edge_index.astype(_i32))

    deg0, deg1 = _deg_pass(idx2)
    degsp = jnp.concatenate([deg0 + deg1,
                             jnp.zeros((NPOOL - NPAD,), _f32)])
    dk = [degsp[k::4].reshape(GRID, 1, PBLK) for k in range(4)]
    batchp = jnp.pad(batch.astype(_i32), (0, NPOOL - N), constant_values=B)
    bk = [batchp[k::4].reshape(GRID, 1, PBLK) for k in range(4)]

    emb_out = _emb_call(smiles_embedding, W_e1, b_e1, W_e2, b_e2)

    x_p = x.reshape(N // 4, 256)
    eye4 = jnp.eye(4, dtype=_f32)
    W4lo = jnp.kron(eye4, W_c1[:, :32])
    W4hi = jnp.kron(eye4, W_c1[:, 32:])
    y1lo_p, y1hi_p = _y1_call(dk, x_p, W4lo, W4hi)
    s1lo, s1hi = _conv_pass(y1lo_p.reshape(YROWS, 32),
                            y1hi_p.reshape(YROWS, 32), idx2)

    bplo = jnp.tile(b_c1[:32], 4).reshape(1, 128)
    bphi = jnp.tile(b_c1[32:], 4).reshape(1, 128)
    y2lo_p, y2hi_p = _y2_call(dk, s1lo.reshape(NP4, 128),
                              s1hi.reshape(NP4, 128), y1lo_p, y1hi_p,
                              bplo, bphi)
    s2lo, s2hi = _conv_pass(y2lo_p.reshape(YROWS, 32),
                            y2hi_p.reshape(YROWS, 32), idx2)

    return _head_call(dk, bk, s2lo.reshape(NP4, 128), s2hi.reshape(NP4, 128),
                      y2lo_p, y2hi_p, emb_out,
                      W_c2, b_c2, W_gf, b_gf, W_a1, b_a1, W_a2, b_a2,
                      W_f1, b_f1, W_ff, b_ff)
